# baseline (speedup 1.0000x reference)
import functools

import jax
import jax.numpy as jnp
from jax import lax
from jax.experimental import pallas as pl
from jax.experimental.pallas import tpu as pltpu
from jax.experimental.pallas import tpu_sc as plsc

VOCAB = 1000000
DIM = 64
SEQ = 200
BATCH = 4096

NC = 2
NS = 16
LANES = 16
NW = NC * NS
COLS = BATCH // NW

VB = 16384


def _proj_body(tblt_ref, w_ref, b_ref, out_ref):
    pid = pl.program_id(0)
    blk = tblt_ref[...]
    w = w_ref[...]
    prod = lax.dot_general(w, blk, (((1,), (0,)), ((), ())),
                           preferred_element_type=jnp.float32)
    rows = lax.broadcasted_iota(jnp.int32, (1, VB), 1) + pid * VB
    out_ref[...] = jnp.where(rows == 0, 0.0, prod) + b_ref[0, 0]


def _make_proj(table, w, b):
    grid = (VOCAB + VB - 1) // VB
    proj2d = pl.pallas_call(
        _proj_body,
        grid=(grid,),
        in_specs=[
            pl.BlockSpec((DIM, VB), lambda i: (0, i)),
            pl.BlockSpec((1, DIM), lambda i: (0, 0)),
            pl.BlockSpec((1, 1), lambda i: (0, 0)),
        ],
        out_specs=pl.BlockSpec((1, VB), lambda i: (0, i)),
        out_shape=jax.ShapeDtypeStruct((1, VOCAB), jnp.float32),
    )(table.T, w, b)
    return proj2d.reshape(VOCAB)


_SC_MESH = plsc.VectorSubcoreMesh(core_axis_name="c", subcore_axis_name="s")


@functools.partial(
    pl.kernel,
    out_type=jax.ShapeDtypeStruct((BATCH,), jnp.float32),
    mesh=_SC_MESH,
    scratch_types=[
        pltpu.VMEM((SEQ, COLS), jnp.int32),
        pltpu.VMEM((SEQ, COLS), jnp.float32),
        pltpu.VMEM((COLS,), jnp.float32),
        pltpu.SemaphoreType.DMA,
    ],
)
def _sc_pool(text_hbm, proj_hbm, out_hbm, idx_v, vals_v, out_v, sem):
    wid = lax.axis_index("s") * NC + lax.axis_index("c")
    base = wid * COLS
    pltpu.sync_copy(text_hbm.at[:, pl.ds(base, COLS)], idx_v)
    NG = SEQ // 8
    NJ = COLS // LANES

    def issue(g):
        for u in range(8):
            s = g * 8 + u
            pltpu.async_copy(proj_hbm.at[idx_v.at[s]], vals_v.at[s], sem)

    issue(0)

    def body(g, accs):
        @pl.when(g < NG - 1)
        def _():
            issue_g = g + 1
            for u in range(8):
                s = issue_g * 8 + u
                pltpu.async_copy(proj_hbm.at[idx_v.at[s]], vals_v.at[s], sem)
        for u in range(8):
            pltpu.make_async_copy(
                proj_hbm.at[idx_v.at[0]], vals_v.at[0], sem).wait()
        new = []
        for jv in range(NJ):
            a = accs[jv]
            for u in range(8):
                a = a + vals_v[g * 8 + u, pl.ds(jv * LANES, LANES)]
            new.append(a)
        return tuple(new)

    accs = lax.fori_loop(
        0, NG, body, tuple(jnp.zeros((LANES,), jnp.float32) for _ in range(NJ)))
    inv = jnp.float32(1.0 / SEQ)
    for jv in range(NJ):
        out_v[pl.ds(jv * LANES, LANES)] = accs[jv] * inv
    pltpu.sync_copy(out_v, out_hbm.at[pl.ds(base, COLS)])


def kernel(text, text_len, table, W, b):
    del text_len
    text = text.astype(jnp.int32)
    proj = _make_proj(table, W, b.reshape(1, 1))
    out = _sc_pool(text, proj)
    return out.reshape(BATCH, 1)


# speedup vs baseline: 1.0762x; 1.0762x over previous
import functools

import jax
import jax.numpy as jnp
from jax import lax
from jax.experimental import pallas as pl
from jax.experimental.pallas import tpu as pltpu
from jax.experimental.pallas import tpu_sc as plsc

VOCAB = 1000000
DIM = 64
SEQ = 200
BATCH = 4096

NC = 2
NS = 16
LANES = 16
NW = NC * NS
COLS = BATCH // NW

VB = 32768


def _proj_body(tblt_ref, w_ref, b_ref, out_ref):
    pid = pl.program_id(0)
    blk = tblt_ref[...]
    w = w_ref[...]
    prod = lax.dot_general(w, blk, (((1,), (0,)), ((), ())),
                           preferred_element_type=jnp.float32)
    rows = lax.broadcasted_iota(jnp.int32, (1, VB), 1) + pid * VB
    out_ref[...] = jnp.where(rows == 0, 0.0, prod) + b_ref[0, 0]


def _make_proj(table, w, b):
    grid = (VOCAB + VB - 1) // VB
    proj2d = pl.pallas_call(
        _proj_body,
        grid=(grid,),
        in_specs=[
            pl.BlockSpec((DIM, VB), lambda i: (0, i)),
            pl.BlockSpec((1, DIM), lambda i: (0, 0)),
            pl.BlockSpec((1, 1), lambda i: (0, 0)),
        ],
        out_specs=pl.BlockSpec((1, VB), lambda i: (0, i)),
        out_shape=jax.ShapeDtypeStruct((1, VOCAB), jnp.float32),
    )(table.T, w, b)
    return proj2d.reshape(VOCAB)


_SC_MESH = plsc.VectorSubcoreMesh(core_axis_name="c", subcore_axis_name="s")


@functools.partial(
    pl.kernel,
    out_type=jax.ShapeDtypeStruct((BATCH,), jnp.float32),
    mesh=_SC_MESH,
    scratch_types=[
        pltpu.VMEM((SEQ, COLS), jnp.int32),
        pltpu.VMEM((SEQ, COLS), jnp.float32),
        pltpu.VMEM((COLS,), jnp.float32),
        pltpu.SemaphoreType.DMA,
    ],
)
def _sc_pool(text_hbm, proj_hbm, out_hbm, idx_v, vals_v, out_v, sem):
    wid = lax.axis_index("s") * NC + lax.axis_index("c")
    base = wid * COLS
    pltpu.sync_copy(text_hbm.at[:, pl.ds(base, COLS)], idx_v)
    NG = SEQ // 8
    NJ = COLS // LANES

    def issue(g):
        for u in range(8):
            s = g * 8 + u
            pltpu.async_copy(proj_hbm.at[idx_v.at[s]], vals_v.at[s], sem)

    issue(0)
    issue(1)

    def body(g, accs):
        @pl.when(g < NG - 2)
        def _():
            issue_g = g + 2
            for u in range(8):
                s = issue_g * 8 + u
                pltpu.async_copy(proj_hbm.at[idx_v.at[s]], vals_v.at[s], sem)
        for u in range(8):
            pltpu.make_async_copy(
                proj_hbm.at[idx_v.at[0]], vals_v.at[0], sem).wait()
        new = []
        for jv in range(NJ):
            a = accs[jv]
            for u in range(8):
                a = a + vals_v[g * 8 + u, pl.ds(jv * LANES, LANES)]
            new.append(a)
        return tuple(new)

    accs = lax.fori_loop(
        0, NG, body, tuple(jnp.zeros((LANES,), jnp.float32) for _ in range(NJ)))
    inv = jnp.float32(1.0 / SEQ)
    for jv in range(NJ):
        out_v[pl.ds(jv * LANES, LANES)] = accs[jv] * inv
    pltpu.sync_copy(out_v, out_hbm.at[pl.ds(base, COLS)])


def kernel(text, text_len, table, W, b):
    del text_len
    text = text.astype(jnp.int32)
    proj = _make_proj(table, W, b.reshape(1, 1))
    out = _sc_pool(text, proj)
    return out.reshape(BATCH, 1)


# speedup vs baseline: 1.7504x; 1.6265x over previous
import functools

import jax
import jax.numpy as jnp
from jax import lax
from jax.experimental import pallas as pl
from jax.experimental.pallas import tpu as pltpu
from jax.experimental.pallas import tpu_sc as plsc

VOCAB = 1000000
DIM = 64
SEQ = 200
BATCH = 4096

NC = 2
NS = 16
LANES = 16
NW = NC * NS
COLS = BATCH // NW

VB = 32768
NGRID = (VOCAB + VB - 1) // VB
VSPAN = NGRID * VB
SEG = VSPAN // NS


def _proj_body(tblt_ref, w_ref, b_ref, out_ref):
    pid = pl.program_id(0)
    blk = tblt_ref[...]
    w = w_ref[...]
    prod = lax.dot_general(w, blk, (((1,), (0,)), ((), ())),
                           preferred_element_type=jnp.float32)
    rows = lax.broadcasted_iota(jnp.int32, (1, VB), 1) + pid * VB
    out_ref[...] = jnp.where(rows == 0, 0.0, prod) + b_ref[0, 0]


def _make_proj(table, w, b):
    proj2d = pl.pallas_call(
        _proj_body,
        grid=(NGRID,),
        in_specs=[
            pl.BlockSpec((DIM, VB), lambda i: (0, i)),
            pl.BlockSpec((1, DIM), lambda i: (0, 0)),
            pl.BlockSpec((1, 1), lambda i: (0, 0)),
        ],
        out_specs=pl.BlockSpec((1, VB), lambda i: (0, i)),
        out_shape=jax.ShapeDtypeStruct((1, VSPAN), jnp.float32),
    )(table.T, w, b)
    return proj2d.reshape(VSPAN)


_SC_MESH = plsc.VectorSubcoreMesh(core_axis_name="c", subcore_axis_name="s")


@functools.partial(
    pl.kernel,
    out_type=jax.ShapeDtypeStruct((BATCH,), jnp.float32),
    mesh=_SC_MESH,
    scratch_types=[
        pltpu.VMEM((SEQ, COLS), jnp.int32),
        pltpu.VMEM((SEQ, COLS), jnp.float32),
        pltpu.VMEM((COLS,), jnp.float32),
        pltpu.VMEM_SHARED((VSPAN,), jnp.float32),
        pltpu.SemaphoreType.DMA,
        pltpu.SemaphoreType.DMA,
    ],
)
def _sc_pool(text_hbm, proj_hbm, out_hbm, idx_v, vals_v, out_v, proj_sh, sem,
             sem2):
    sid = lax.axis_index("s")
    wid = sid * NC + lax.axis_index("c")
    base = wid * COLS
    stage = pltpu.async_copy(
        proj_hbm.at[pl.ds(sid * SEG, SEG)], proj_sh.at[pl.ds(sid * SEG, SEG)],
        sem2)
    pltpu.sync_copy(text_hbm.at[:, pl.ds(base, COLS)], idx_v)
    stage.wait()
    plsc.subcore_barrier()
    NG = SEQ // 8
    NJ = COLS // LANES

    def issue(g):
        for u in range(8):
            s = g * 8 + u
            pltpu.async_copy(proj_sh.at[idx_v.at[s]], vals_v.at[s], sem)

    issue(0)
    issue(1)

    def body(g, accs):
        @pl.when(g < NG - 2)
        def _():
            issue_g = g + 2
            for u in range(8):
                s = issue_g * 8 + u
                pltpu.async_copy(proj_sh.at[idx_v.at[s]], vals_v.at[s], sem)
        for u in range(8):
            pltpu.make_async_copy(
                proj_hbm.at[idx_v.at[0]], vals_v.at[0], sem).wait()
        new = []
        for jv in range(NJ):
            a = accs[jv]
            for u in range(8):
                a = a + vals_v[g * 8 + u, pl.ds(jv * LANES, LANES)]
            new.append(a)
        return tuple(new)

    accs = lax.fori_loop(
        0, NG, body, tuple(jnp.zeros((LANES,), jnp.float32) for _ in range(NJ)))
    inv = jnp.float32(1.0 / SEQ)
    for jv in range(NJ):
        out_v[pl.ds(jv * LANES, LANES)] = accs[jv] * inv
    pltpu.sync_copy(out_v, out_hbm.at[pl.ds(base, COLS)])


def kernel(text, text_len, table, W, b):
    del text_len
    text = text.astype(jnp.int32)
    proj = _make_proj(table, W, b.reshape(1, 1))
    out = _sc_pool(text, proj)
    return out.reshape(BATCH, 1)


# speedup vs baseline: 1.7543x; 1.0022x over previous
import functools

import jax
import jax.numpy as jnp
from jax import lax
from jax.experimental import pallas as pl
from jax.experimental.pallas import tpu as pltpu
from jax.experimental.pallas import tpu_sc as plsc

VOCAB = 1000000
DIM = 64
SEQ = 200
BATCH = 4096

NC = 2
NS = 16
LANES = 16
NW = NC * NS
COLS = BATCH // NW

VB = 32768
NGRID = (VOCAB + VB - 1) // VB
VSPAN = NGRID * VB
SEG = VSPAN // NS


def _proj_body(tblt_ref, w_ref, b_ref, out_ref):
    pid = pl.program_id(0)
    blk = tblt_ref[...]
    w = w_ref[...]
    prod = lax.dot_general(w, blk, (((1,), (0,)), ((), ())),
                           preferred_element_type=jnp.float32)
    rows = lax.broadcasted_iota(jnp.int32, (1, VB), 1) + pid * VB
    out_ref[...] = jnp.where(rows == 0, 0.0, prod) + b_ref[0, 0]


def _make_proj(table, w, b):
    proj2d = pl.pallas_call(
        _proj_body,
        grid=(NGRID,),
        in_specs=[
            pl.BlockSpec((DIM, VB), lambda i: (0, i)),
            pl.BlockSpec((1, DIM), lambda i: (0, 0)),
            pl.BlockSpec((1, 1), lambda i: (0, 0)),
        ],
        out_specs=pl.BlockSpec((1, VB), lambda i: (0, i)),
        out_shape=jax.ShapeDtypeStruct((1, VSPAN), jnp.float32),
    )(table.T, w, b)
    return proj2d.reshape(VSPAN)


_SC_MESH = plsc.VectorSubcoreMesh(core_axis_name="c", subcore_axis_name="s")


@functools.partial(
    pl.kernel,
    out_type=jax.ShapeDtypeStruct((BATCH,), jnp.float32),
    mesh=_SC_MESH,
    scratch_types=[
        pltpu.VMEM((SEQ, COLS), jnp.int32),
        pltpu.VMEM((SEQ, COLS), jnp.float32),
        pltpu.VMEM((COLS,), jnp.float32),
        pltpu.VMEM_SHARED((VSPAN,), jnp.float32),
        pltpu.SemaphoreType.DMA,
        pltpu.SemaphoreType.DMA,
    ],
)
def _sc_pool(text_hbm, proj_hbm, out_hbm, idx_v, vals_v, out_v, proj_sh, sem,
             sem2):
    sid = lax.axis_index("s")
    wid = sid * NC + lax.axis_index("c")
    base = wid * COLS
    stage = pltpu.async_copy(
        proj_hbm.at[pl.ds(sid * SEG, SEG)], proj_sh.at[pl.ds(sid * SEG, SEG)],
        sem2)
    pltpu.sync_copy(text_hbm.at[:, pl.ds(base, COLS)], idx_v)
    stage.wait()
    plsc.subcore_barrier()
    NG = SEQ // 8
    NJ = COLS // LANES

    def issue(g):
        for u in range(8):
            s = g * 8 + u
            pltpu.async_copy(proj_sh.at[idx_v.at[s]], vals_v.at[s], sem)

    issue(0)
    issue(1)
    issue(2)

    def body(g, accs):
        @pl.when(g < NG - 3)
        def _():
            issue_g = g + 3
            for u in range(8):
                s = issue_g * 8 + u
                pltpu.async_copy(proj_sh.at[idx_v.at[s]], vals_v.at[s], sem)
        for u in range(8):
            pltpu.make_async_copy(
                proj_hbm.at[idx_v.at[0]], vals_v.at[0], sem).wait()
        new = []
        for jv in range(NJ):
            a = accs[jv]
            for u in range(8):
                a = a + vals_v[g * 8 + u, pl.ds(jv * LANES, LANES)]
            new.append(a)
        return tuple(new)

    accs = lax.fori_loop(
        0, NG, body, tuple(jnp.zeros((LANES,), jnp.float32) for _ in range(NJ)))
    inv = jnp.float32(1.0 / SEQ)
    for jv in range(NJ):
        out_v[pl.ds(jv * LANES, LANES)] = accs[jv] * inv
    pltpu.sync_copy(out_v, out_hbm.at[pl.ds(base, COLS)])


def kernel(text, text_len, table, W, b):
    del text_len
    text = text.astype(jnp.int32)
    proj = _make_proj(table, W, b.reshape(1, 1))
    out = _sc_pool(text, proj)
    return out.reshape(BATCH, 1)
